# Initial kernel scaffold; baseline (speedup 1.0000x reference)
import jax
import jax.numpy as jnp
from jax.experimental import pallas as pl


def kernel(x, table):
    raise NotImplementedError("write your pallas kernel here")


# speedup vs baseline: 1.4983x; 1.4983x over previous
import functools

import jax
import jax.numpy as jnp
from jax import lax
from jax.experimental import pallas as pl
from jax.experimental.pallas import tpu as pltpu
from jax.experimental.pallas import tpu_sc as plsc

_DIM = 32
_NC = 2
_NS = 16
_NW = _NC * _NS
_K = 128
_NBUF = 8
_DELTA = 4


@functools.lru_cache(maxsize=None)
def _build(batch, vocab):
    assert batch % (_NW * _K) == 0
    bpw = batch // _NW
    nch = bpw // _K
    assert (nch - _NBUF) % _NBUF == 0
    mesh = plsc.VectorSubcoreMesh(core_axis_name="c", subcore_axis_name="s")

    @functools.partial(
        pl.kernel,
        mesh=mesh,
        out_type=jax.ShapeDtypeStruct((_NW, nch, _K, _DIM), jnp.float32),
        scratch_types=(
            [pltpu.VMEM((nch, _K), jnp.int32),
             pltpu.VMEM((_NBUF, _K, _DIM), jnp.float32)]
            + [pltpu.SemaphoreType.DMA] * (2 * _NBUF)
        ),
        compiler_params=pltpu.CompilerParams(use_tc_tiling_on_sc=False),
    )
    def emb(idx_hbm, table_hbm, out_hbm, idx_v, rows_v, *sems):
        gsem = sems[:_NBUF]
        osem = sems[_NBUF:]
        wid = lax.axis_index("s") * _NC + lax.axis_index("c")
        pltpu.sync_copy(idx_hbm.at[wid], idx_v)

        def start_gather(c, b):
            pltpu.async_copy(table_hbm.at[idx_v.at[c]], rows_v.at[b], gsem[b])

        def wait_gather(c, b):
            pltpu.make_async_copy(
                table_hbm.at[idx_v.at[c]], rows_v.at[b], gsem[b]).wait()

        def start_out(c, b):
            pltpu.async_copy(rows_v.at[b], out_hbm.at[wid, c], osem[b])

        def wait_out(c, b):
            pltpu.make_async_copy(
                rows_v.at[b], out_hbm.at[wid, c], osem[b]).wait()

        for g in range(_DELTA):
            start_gather(g, g % _NBUF)
        for g in range(_DELTA, _NBUF):
            start_gather(g, g % _NBUF)
            c = g - _DELTA
            wait_gather(c, c % _NBUF)
            start_out(c, c % _NBUF)

        def group(i, carry):
            g0 = _NBUF + i * _NBUF
            for b in range(_NBUF):
                g = g0 + b
                wait_out(g - _NBUF, b)
                start_gather(g, b)
                cb = (b + _NBUF - _DELTA) % _NBUF
                wait_gather(g - _DELTA, cb)
                start_out(g - _DELTA, cb)
            return carry

        lax.fori_loop(0, (nch - _NBUF) // _NBUF, group, 0)

        for c in range(nch - _DELTA, nch):
            wait_gather(c, c % _NBUF)
            start_out(c, c % _NBUF)
        for c in range(nch - _NBUF, nch):
            wait_out(c, c % _NBUF)

    return emb


def kernel(x, table):
    n, m = x.shape
    batch = n * m
    idx = x.reshape(_NW, batch // _NW // _K, _K).astype(jnp.int32)
    out = _build(batch, table.shape[0])(idx, table)
    return out.reshape(n, m, _DIM)


# speedup vs baseline: 1.5844x; 1.0575x over previous
import functools

import jax
import jax.numpy as jnp
from jax import lax
from jax.experimental import pallas as pl
from jax.experimental.pallas import tpu as pltpu
from jax.experimental.pallas import tpu_sc as plsc

_DIM = 32
_NC = 2
_NS = 16
_NW = _NC * _NS
_K = 128
_NBUF = 8
_DELTA = 4


@functools.lru_cache(maxsize=None)
def _build(batch, vocab):
    assert batch % (_NW * _K) == 0
    bpw = batch // _NW
    nch = bpw // _K
    assert (nch - _NBUF) % _NBUF == 0
    mesh = plsc.VectorSubcoreMesh(core_axis_name="c", subcore_axis_name="s")

    @functools.partial(
        pl.kernel,
        mesh=mesh,
        out_type=jax.ShapeDtypeStruct((_NW, nch, _K, _DIM), jnp.float32),
        scratch_types=(
            [pltpu.VMEM((nch, _K), jnp.int32),
             pltpu.VMEM((_NBUF, _K, _DIM), jnp.float32)]
            + [pltpu.SemaphoreType.DMA] * (2 * _NBUF)
        ),
        compiler_params=pltpu.CompilerParams(use_tc_tiling_on_sc=False),
    )
    def emb(idx_hbm, table_hbm, out_hbm, idx_v, rows_v, *sems):
        gsem = sems[:_NBUF]
        osem = sems[_NBUF:]
        wid = lax.axis_index("s") * _NC + lax.axis_index("c")
        pltpu.sync_copy(idx_hbm.at[wid], idx_v)

        def start_gather(c, b):
            pltpu.async_copy(table_hbm.at[idx_v.at[c]], rows_v.at[b], gsem[b])

        def wait_gather(c, b):
            pltpu.make_async_copy(
                table_hbm.at[idx_v.at[c]], rows_v.at[b], gsem[b]).wait()

        def start_out(c, b):
            pltpu.async_copy(rows_v.at[b], out_hbm.at[wid, c], osem[b])

        def wait_out(c, b):
            pltpu.make_async_copy(
                rows_v.at[b], out_hbm.at[wid, c], osem[b]).wait()

        for g in range(_DELTA):
            start_gather(g, g % _NBUF)
        for g in range(_DELTA, _NBUF):
            start_gather(g, g % _NBUF)
            c = g - _DELTA
            wait_gather(c, c % _NBUF)
            start_out(c, c % _NBUF)

        def group(i, carry):
            g0 = _NBUF + i * _NBUF
            for b in range(_NBUF):
                g = g0 + b
                wait_out(g - _NBUF, b)
                start_gather(g, b)
                cb = (b + _NBUF - _DELTA) % _NBUF
                wait_gather(g - _DELTA, cb)
                start_out(g - _DELTA, cb)
            return carry

        lax.fori_loop(0, (nch - _NBUF) // _NBUF, group, 0)

        for c in range(nch - _DELTA, nch):
            wait_gather(c, c % _NBUF)
            start_out(c, c % _NBUF)
        for c in range(nch - _NBUF, nch):
            wait_out(c, c % _NBUF)

    return emb


def _transpose_tc(src, bi, bj):
    rows, cols = src.shape
    gi = -(-rows // bi)
    gj = -(-cols // bj)

    def body(in_ref, out_ref):
        out_ref[...] = in_ref[...].T

    return pl.pallas_call(
        body,
        grid=(gi, gj),
        in_specs=[pl.BlockSpec((bi, bj), lambda i, j: (i, j))],
        out_specs=pl.BlockSpec((bj, bi), lambda i, j: (j, i)),
        out_shape=jax.ShapeDtypeStruct((cols, rows), src.dtype),
    )(src)


def kernel(x, table):
    n, m = x.shape
    batch = n * m
    idx = x.reshape(_NW, batch // _NW // _K, _K).astype(jnp.int32)
    table_rm = _transpose_tc(jnp.swapaxes(table, 0, 1), _DIM, 4096)
    out = _build(batch, table.shape[0])(idx, table_rm)
    out_t = _transpose_tc(out.reshape(n, m * _DIM), 512, 640)
    return jnp.transpose(out_t.reshape(m, _DIM, n), (2, 0, 1))
